# baseline (speedup 1.0000x reference)
import functools

import jax
import jax.numpy as jnp
from jax import lax
from jax.experimental import pallas as pl
from jax.experimental.pallas import tpu as pltpu
from jax.experimental.pallas import tpu_sc as plsc

N_CORES = 2
N_SUBCORES = 16
NW = N_CORES * N_SUBCORES
CHUNK = 128
TBLK = 8192


def _transpose_body(t_ref, o_ref):
    eye = jnp.eye(t_ref.shape[0], dtype=jnp.float32)
    o_ref[...] = jax.lax.dot_general(
        t_ref[...], eye, (((0,), (0,)), ((), ())),
        preferred_element_type=jnp.float32,
        precision=jax.lax.Precision.HIGHEST)


def _relayout(table_t, vocab, dim):
    grid = (vocab + TBLK - 1) // TBLK
    return pl.pallas_call(
        _transpose_body,
        grid=(grid,),
        in_specs=[pl.BlockSpec((dim, TBLK), lambda i: (0, i))],
        out_specs=pl.BlockSpec((TBLK, dim), lambda i: (i, 0)),
        out_shape=jax.ShapeDtypeStruct((vocab, dim), jnp.float32),
    )(table_t)


def _score_body(u_ref, i_ref, o_ref):
    o_ref[...] = jnp.sum(u_ref[...] * i_ref[...], axis=1, keepdims=True)


def _sc_gather(user_table, item_table, users2d, items2d, batch, dim):
    b_per_w = batch // NW
    n_chunks = b_per_w // CHUNK
    mesh = plsc.VectorSubcoreMesh(core_axis_name="c", subcore_axis_name="s")

    @functools.partial(
        pl.kernel,
        mesh=mesh,
        out_type=[
            jax.ShapeDtypeStruct((batch, dim), jnp.float32),
            jax.ShapeDtypeStruct((batch, dim), jnp.float32),
        ],
        scratch_types=[
            pltpu.VMEM((n_chunks, CHUNK), jnp.int32),
            pltpu.VMEM((n_chunks, CHUNK), jnp.int32),
            pltpu.VMEM((b_per_w, dim), jnp.float32),
            pltpu.VMEM((b_per_w, dim), jnp.float32),
            pltpu.SemaphoreType.DMA,
        ],
        compiler_params=pltpu.CompilerParams(use_tc_tiling_on_sc=False),
    )
    def gather_kernel(u_tab, i_tab, u_idx_hbm, i_idx_hbm, u_out, i_out,
                      u_idx, i_idx, u_rows, i_rows, sem):
        wid = lax.axis_index("s") * N_CORES + lax.axis_index("c")
        base = wid * b_per_w
        row0 = wid * n_chunks
        pltpu.sync_copy(u_idx_hbm.at[pl.ds(row0, n_chunks)], u_idx)
        pltpu.sync_copy(i_idx_hbm.at[pl.ds(row0, n_chunks)], i_idx)
        copies = []
        for c in range(n_chunks):
            copies.append(pltpu.async_copy(
                u_tab.at[u_idx.at[c]], u_rows.at[pl.ds(c * CHUNK, CHUNK)], sem))
            copies.append(pltpu.async_copy(
                i_tab.at[i_idx.at[c]], i_rows.at[pl.ds(c * CHUNK, CHUNK)], sem))
        for cp in copies:
            cp.wait()
        pltpu.sync_copy(u_rows, u_out.at[pl.ds(base, b_per_w)])
        pltpu.sync_copy(i_rows, i_out.at[pl.ds(base, b_per_w)])

    return gather_kernel(user_table, item_table, users2d, items2d)


def kernel(users, items, user_table, item_table):
    batch = users.shape[0]
    vocab, dim = user_table.shape
    users2d = users.astype(jnp.int32).reshape(batch // CHUNK, CHUNK)
    items2d = items.astype(jnp.int32).reshape(batch // CHUNK, CHUNK)
    ut_rm = _relayout(user_table.T, vocab, dim)
    it_rm = _relayout(item_table.T, vocab, dim)
    user_emb, item_emb = _sc_gather(
        ut_rm, it_rm, users2d, items2d, batch, dim)
    scores2d = pl.pallas_call(
        _score_body,
        out_shape=jax.ShapeDtypeStruct((batch, 1), jnp.float32),
    )(user_emb, item_emb)
    return user_emb, item_emb, scores2d.reshape(batch)


# speedup vs baseline: 1.4203x; 1.4203x over previous
import functools

import jax
import jax.numpy as jnp
from jax import lax
from jax.experimental import pallas as pl
from jax.experimental.pallas import tpu as pltpu
from jax.experimental.pallas import tpu_sc as plsc

N_CORES = 2
N_SUBCORES = 16
NW = N_CORES * N_SUBCORES
CHUNK = 128
TBLK = 32768


def _transpose_body(t_ref, o_ref):
    o_ref[...] = t_ref[...].T


def _relayout(table_t, vocab, dim):
    grid = (vocab + TBLK - 1) // TBLK
    return pl.pallas_call(
        _transpose_body,
        grid=(grid,),
        in_specs=[pl.BlockSpec((dim, TBLK), lambda i: (0, i))],
        out_specs=pl.BlockSpec((TBLK, dim), lambda i: (i, 0)),
        out_shape=jax.ShapeDtypeStruct((vocab, dim), jnp.float32),
    )(table_t)


def _score_body(u_ref, i_ref, o_ref):
    o_ref[...] = jnp.sum(u_ref[...] * i_ref[...], axis=1, keepdims=True)


def _sc_gather(user_table, item_table, users2d, items2d, batch, dim):
    b_per_w = batch // NW
    n_chunks = b_per_w // CHUNK
    mesh = plsc.VectorSubcoreMesh(core_axis_name="c", subcore_axis_name="s")

    @functools.partial(
        pl.kernel,
        mesh=mesh,
        out_type=[
            jax.ShapeDtypeStruct((batch, dim), jnp.float32),
            jax.ShapeDtypeStruct((batch, dim), jnp.float32),
        ],
        scratch_types=[
            pltpu.VMEM((n_chunks, CHUNK), jnp.int32),
            pltpu.VMEM((n_chunks, CHUNK), jnp.int32),
            pltpu.VMEM((b_per_w, dim), jnp.float32),
            pltpu.VMEM((b_per_w, dim), jnp.float32),
            pltpu.SemaphoreType.DMA,
        ],
        compiler_params=pltpu.CompilerParams(use_tc_tiling_on_sc=False),
    )
    def gather_kernel(u_tab, i_tab, u_idx_hbm, i_idx_hbm, u_out, i_out,
                      u_idx, i_idx, u_rows, i_rows, sem):
        wid = lax.axis_index("s") * N_CORES + lax.axis_index("c")
        base = wid * b_per_w
        row0 = wid * n_chunks
        pltpu.sync_copy(u_idx_hbm.at[pl.ds(row0, n_chunks)], u_idx)
        pltpu.sync_copy(i_idx_hbm.at[pl.ds(row0, n_chunks)], i_idx)
        copies = []
        for c in range(n_chunks):
            copies.append(pltpu.async_copy(
                u_tab.at[u_idx.at[c]], u_rows.at[pl.ds(c * CHUNK, CHUNK)], sem))
            copies.append(pltpu.async_copy(
                i_tab.at[i_idx.at[c]], i_rows.at[pl.ds(c * CHUNK, CHUNK)], sem))
        for cp in copies:
            cp.wait()
        pltpu.sync_copy(u_rows, u_out.at[pl.ds(base, b_per_w)])
        pltpu.sync_copy(i_rows, i_out.at[pl.ds(base, b_per_w)])

    return gather_kernel(user_table, item_table, users2d, items2d)


def kernel(users, items, user_table, item_table):
    batch = users.shape[0]
    vocab, dim = user_table.shape
    users2d = users.astype(jnp.int32).reshape(batch // CHUNK, CHUNK)
    items2d = items.astype(jnp.int32).reshape(batch // CHUNK, CHUNK)
    ut_rm = _relayout(user_table.T, vocab, dim)
    it_rm = _relayout(item_table.T, vocab, dim)
    user_emb, item_emb = _sc_gather(
        ut_rm, it_rm, users2d, items2d, batch, dim)
    scores2d = pl.pallas_call(
        _score_body,
        out_shape=jax.ShapeDtypeStruct((batch, 1), jnp.float32),
    )(user_emb, item_emb)
    return user_emb, item_emb, scores2d.reshape(batch)


# speedup vs baseline: 2.9468x; 2.0747x over previous
import functools

import jax
import jax.numpy as jnp
from jax import lax
from jax.experimental import pallas as pl
from jax.experimental.pallas import tpu as pltpu
from jax.experimental.pallas import tpu_sc as plsc

N_CORES = 2
N_SUBCORES = 16
NW = N_CORES * N_SUBCORES
LANES = 16
CHUNK = 128
TBLK = 32768
SUB = TBLK // 4


def _transpose_body(t_ref, o_ref):
    x = t_ref[...]
    for a in range(4):
        o_ref[:, a * 32:(a + 1) * 32] = x[:, a * SUB:(a + 1) * SUB].T


def _relayout(table_t, vocab, dim):
    grid = (vocab + TBLK - 1) // TBLK
    packed = pl.pallas_call(
        _transpose_body,
        grid=(grid,),
        in_specs=[pl.BlockSpec((dim, TBLK), lambda i: (0, i))],
        out_specs=pl.BlockSpec((SUB, 4 * dim), lambda i: (i, 0)),
        out_shape=jax.ShapeDtypeStruct((grid * SUB, 4 * dim), jnp.float32),
    )(table_t)
    return packed.reshape(4 * grid * SUB, dim)


def _score_body(u_ref, i_ref, o_ref):
    o_ref[...] = jnp.sum(u_ref[...] * i_ref[...], axis=1, keepdims=True)


def _sc_gather(user_pk, item_pk, users2d, items2d, batch, dim):
    b_per_w = batch // NW
    n_chunks = b_per_w // CHUNK
    mesh = plsc.VectorSubcoreMesh(core_axis_name="c", subcore_axis_name="s")

    @functools.partial(
        pl.kernel,
        mesh=mesh,
        out_type=[
            jax.ShapeDtypeStruct((batch, dim), jnp.float32),
            jax.ShapeDtypeStruct((batch, dim), jnp.float32),
        ],
        scratch_types=[
            pltpu.VMEM((n_chunks, CHUNK), jnp.int32),
            pltpu.VMEM((n_chunks, CHUNK), jnp.int32),
            pltpu.VMEM((n_chunks, CHUNK), jnp.int32),
            pltpu.VMEM((n_chunks, CHUNK), jnp.int32),
            pltpu.VMEM((b_per_w, dim), jnp.float32),
            pltpu.VMEM((b_per_w, dim), jnp.float32),
            pltpu.SemaphoreType.DMA,
        ],
        compiler_params=pltpu.CompilerParams(use_tc_tiling_on_sc=False),
    )
    def gather_kernel(u_tab, i_tab, u_idx_hbm, i_idx_hbm, u_out, i_out,
                      u_idx, i_idx, u_f, i_f, u_rows, i_rows, sem):
        wid = lax.axis_index("s") * N_CORES + lax.axis_index("c")
        base = wid * b_per_w
        row0 = wid * n_chunks
        pltpu.sync_copy(u_idx_hbm.at[pl.ds(row0, n_chunks)], u_idx)
        pltpu.sync_copy(i_idx_hbm.at[pl.ds(row0, n_chunks)], i_idx)

        for c in range(n_chunks):
            @pl.loop(0, CHUNK, step=LANES)
            def _(j, c=c):
                ru = u_idx[c, pl.ds(j, LANES)]
                u_f[c, pl.ds(j, LANES)] = (
                    ((ru >> 15) << 15) + ((ru & (SUB - 1)) << 2)
                    + ((ru >> 13) & 3))
                ri = i_idx[c, pl.ds(j, LANES)]
                i_f[c, pl.ds(j, LANES)] = (
                    ((ri >> 15) << 15) + ((ri & (SUB - 1)) << 2)
                    + ((ri >> 13) & 3))

        copies = []
        for c in range(n_chunks):
            copies.append(pltpu.async_copy(
                u_tab.at[u_f.at[c]], u_rows.at[pl.ds(c * CHUNK, CHUNK)], sem))
            copies.append(pltpu.async_copy(
                i_tab.at[i_f.at[c]], i_rows.at[pl.ds(c * CHUNK, CHUNK)], sem))
        for cp in copies:
            cp.wait()
        pltpu.sync_copy(u_rows, u_out.at[pl.ds(base, b_per_w)])
        pltpu.sync_copy(i_rows, i_out.at[pl.ds(base, b_per_w)])

    return gather_kernel(user_pk, item_pk, users2d, items2d)


def kernel(users, items, user_table, item_table):
    batch = users.shape[0]
    vocab, dim = user_table.shape
    users2d = users.astype(jnp.int32).reshape(batch // CHUNK, CHUNK)
    items2d = items.astype(jnp.int32).reshape(batch // CHUNK, CHUNK)
    ut_pk = _relayout(user_table.T, vocab, dim)
    it_pk = _relayout(item_table.T, vocab, dim)
    user_emb, item_emb = _sc_gather(
        ut_pk, it_pk, users2d, items2d, batch, dim)
    scores2d = pl.pallas_call(
        _score_body,
        out_shape=jax.ShapeDtypeStruct((batch, 1), jnp.float32),
    )(user_emb, item_emb)
    return user_emb, item_emb, scores2d.reshape(batch)


# speedup vs baseline: 3.8684x; 1.3128x over previous
import functools

import jax
import jax.numpy as jnp
from jax import lax
from jax.experimental import pallas as pl
from jax.experimental.pallas import tpu as pltpu
from jax.experimental.pallas import tpu_sc as plsc

N_CORES = 2
N_SUBCORES = 16
NW = N_CORES * N_SUBCORES
LANES = 16
CHUNK = 128
TBLK = 32768
SUB = TBLK // 4


def _transpose_body(t_ref, o_ref):
    x = t_ref[...].astype(jnp.bfloat16)
    eye = jnp.eye(32, dtype=jnp.bfloat16)
    for a in range(4):
        o_ref[:, a * 32:(a + 1) * 32] = jax.lax.dot_general(
            x[:, a * SUB:(a + 1) * SUB], eye, (((0,), (0,)), ((), ())),
            preferred_element_type=jnp.float32)


def _relayout(table_t, vocab, dim):
    grid = (vocab + TBLK - 1) // TBLK
    packed = pl.pallas_call(
        _transpose_body,
        grid=(grid,),
        in_specs=[pl.BlockSpec((dim, TBLK), lambda i: (0, i))],
        out_specs=pl.BlockSpec((SUB, 4 * dim), lambda i: (i, 0)),
        out_shape=jax.ShapeDtypeStruct((grid * SUB, 4 * dim), jnp.float32),
    )(table_t)
    return packed.reshape(4 * grid * SUB, dim)


def _score_body(u_ref, i_ref, o_ref):
    o_ref[...] = jnp.sum(u_ref[...] * i_ref[...], axis=1, keepdims=True)


def _sc_gather(user_pk, item_pk, users2d, items2d, batch, dim):
    b_per_w = batch // NW
    n_chunks = b_per_w // CHUNK
    mesh = plsc.VectorSubcoreMesh(core_axis_name="c", subcore_axis_name="s")

    @functools.partial(
        pl.kernel,
        mesh=mesh,
        out_type=[
            jax.ShapeDtypeStruct((batch, dim), jnp.float32),
            jax.ShapeDtypeStruct((batch, dim), jnp.float32),
        ],
        scratch_types=[
            pltpu.VMEM((n_chunks, CHUNK), jnp.int32),
            pltpu.VMEM((n_chunks, CHUNK), jnp.int32),
            pltpu.VMEM((n_chunks, CHUNK), jnp.int32),
            pltpu.VMEM((n_chunks, CHUNK), jnp.int32),
            pltpu.VMEM((b_per_w, dim), jnp.float32),
            pltpu.VMEM((b_per_w, dim), jnp.float32),
            pltpu.SemaphoreType.DMA,
        ],
        compiler_params=pltpu.CompilerParams(use_tc_tiling_on_sc=False),
    )
    def gather_kernel(u_tab, i_tab, u_idx_hbm, i_idx_hbm, u_out, i_out,
                      u_idx, i_idx, u_f, i_f, u_rows, i_rows, sem):
        wid = lax.axis_index("s") * N_CORES + lax.axis_index("c")
        base = wid * b_per_w
        row0 = wid * n_chunks
        pltpu.sync_copy(u_idx_hbm.at[pl.ds(row0, n_chunks)], u_idx)
        pltpu.sync_copy(i_idx_hbm.at[pl.ds(row0, n_chunks)], i_idx)

        for c in range(n_chunks):
            @pl.loop(0, CHUNK, step=LANES)
            def _(j, c=c):
                ru = u_idx[c, pl.ds(j, LANES)]
                u_f[c, pl.ds(j, LANES)] = (
                    ((ru >> 15) << 15) + ((ru & (SUB - 1)) << 2)
                    + ((ru >> 13) & 3))
                ri = i_idx[c, pl.ds(j, LANES)]
                i_f[c, pl.ds(j, LANES)] = (
                    ((ri >> 15) << 15) + ((ri & (SUB - 1)) << 2)
                    + ((ri >> 13) & 3))

        copies = []
        for c in range(n_chunks):
            copies.append(pltpu.async_copy(
                u_tab.at[u_f.at[c]], u_rows.at[pl.ds(c * CHUNK, CHUNK)], sem))
            copies.append(pltpu.async_copy(
                i_tab.at[i_f.at[c]], i_rows.at[pl.ds(c * CHUNK, CHUNK)], sem))
        for cp in copies:
            cp.wait()
        pltpu.sync_copy(u_rows, u_out.at[pl.ds(base, b_per_w)])
        pltpu.sync_copy(i_rows, i_out.at[pl.ds(base, b_per_w)])

    return gather_kernel(user_pk, item_pk, users2d, items2d)


def kernel(users, items, user_table, item_table):
    batch = users.shape[0]
    vocab, dim = user_table.shape
    users2d = users.astype(jnp.int32).reshape(batch // CHUNK, CHUNK)
    items2d = items.astype(jnp.int32).reshape(batch // CHUNK, CHUNK)
    ut_pk = _relayout(user_table.T, vocab, dim)
    it_pk = _relayout(item_table.T, vocab, dim)
    user_emb, item_emb = _sc_gather(
        ut_pk, it_pk, users2d, items2d, batch, dim)
    scores2d = pl.pallas_call(
        _score_body,
        out_shape=jax.ShapeDtypeStruct((batch, 1), jnp.float32),
    )(user_emb, item_emb)
    return user_emb, item_emb, scores2d.reshape(batch)
